# baseline (speedup 1.0000x reference)
import jax
import jax.numpy as jnp
from jax import lax
from jax.experimental import pallas as pl
from jax.experimental.pallas import tpu as pltpu

_NUM_SNPS = 100000
_NUM_IMPACTS = 16
_EMB = 16
_BATCH = 1024

_ROWS = 16
_STEPS = _BATCH // _ROWS
_NBUF = 2
_OUTER = _STEPS // _NBUF
_HALF = _ROWS // 2


def _head_body(emb_ref, wpt_ref, bp_ref, gamma_ref, beta_ref, wsb_ref,
               bsbb_ref, tab_ref):
    h = jnp.dot(emb_ref[...], wpt_ref[...],
                preferred_element_type=jnp.float32) + bp_ref[...]
    mu = jnp.mean(h, axis=-1, keepdims=True)
    var = jnp.mean((h - mu) ** 2, axis=-1, keepdims=True)
    h = (h - mu) / jnp.sqrt(var + 1e-5) * gamma_ref[...] + beta_ref[...]
    h = jnp.maximum(h, 0.0)
    tab_ref[...] = jnp.dot(h, wsb_ref[...],
                           preferred_element_type=jnp.float32) + bsbb_ref[...]


def _dense_body(tab_ref, idx_hbm, x_hbm, o_hbm, idx_v, sb_v, bb_v, xb, ob,
                idxsem, insems, outsems):
    def fetch(s, r, start):
        for h in range(2):
            op = pltpu.make_async_copy(
                x_hbm.at[pl.ds(r + h * _HALF, _HALF), :],
                xb.at[s, pl.ds(h * _HALF, _HALF), :],
                insems.at[s, h])
            op.start() if start else op.wait()

    def put(s, r, start):
        for h in range(2):
            op = pltpu.make_async_copy(
                ob.at[s, pl.ds(h * _HALF, _HALF), :],
                o_hbm.at[pl.ds(r + h * _HALF, _HALF), :],
                outsems.at[s, h])
            op.start() if start else op.wait()

    for s in range(_NBUF):
        fetch(s, s * _ROWS, True)

    cp = pltpu.make_async_copy(idx_hbm, idx_v, idxsem)
    cp.start()
    cp.wait()
    idx = idx_v[...]
    ss = jnp.full(idx.shape, tab_ref[0, 0] * 0.5, jnp.float32)
    bb = jnp.full(idx.shape, tab_ref[0, 1] * 0.5, jnp.float32)
    for k in range(1, _NUM_IMPACTS):
        m = idx == k
        ss = jnp.where(m, tab_ref[k, 0] * 0.5, ss)
        bb = jnp.where(m, tab_ref[k, 1] * 0.5, bb)
    sb_v[...] = jnp.broadcast_to(ss, (_ROWS, _NUM_SNPS))
    bb_v[...] = jnp.broadcast_to(bb, (_ROWS, _NUM_SNPS))

    def outer(o, carry):
        t0 = o * _NBUF
        for s in range(_NBUF):
            t = t0 + s
            r = t * _ROWS
            fetch(s, r, False)

            @pl.when(t >= _NBUF)
            def _():
                put(s, (t - _NBUF) * _ROWS, False)

            xx = xb[s]
            ob[s] = xx + xx * jnp.tanh(xx * sb_v[...] + bb_v[...])
            put(s, r, True)

            @pl.when(t + _NBUF < _STEPS)
            def _():
                fetch(s, (t + _NBUF) * _ROWS, True)
        return carry

    lax.fori_loop(0, _OUTER, outer, 0)

    for s in range(_NBUF):
        put(s, (_STEPS - _NBUF + s) * _ROWS, False)


def kernel(x, impact_indices, emb, Wp, bp, gamma, beta, ws, bs, wb, bb):
    wpt = Wp.T
    wsb = jnp.concatenate([ws, wb], axis=1)
    bsbb = jnp.concatenate([bs, bb]).reshape(1, 2)

    tab = pl.pallas_call(
        _head_body,
        out_shape=jax.ShapeDtypeStruct((_NUM_IMPACTS, 2), jnp.float32),
    )(emb, wpt, bp.reshape(1, _EMB), gamma.reshape(1, _EMB),
      beta.reshape(1, _EMB), wsb, bsbb)

    idx = impact_indices.reshape(1, _NUM_SNPS)

    out = pl.pallas_call(
        _dense_body,
        in_specs=[
            pl.BlockSpec(memory_space=pltpu.SMEM),
            pl.BlockSpec(memory_space=pl.ANY),
            pl.BlockSpec(memory_space=pl.ANY),
        ],
        out_specs=pl.BlockSpec(memory_space=pl.ANY),
        out_shape=jax.ShapeDtypeStruct((_BATCH, _NUM_SNPS), jnp.float32),
        scratch_shapes=[
            pltpu.VMEM((1, _NUM_SNPS), jnp.int32),
            pltpu.VMEM((_ROWS, _NUM_SNPS), jnp.float32),
            pltpu.VMEM((_ROWS, _NUM_SNPS), jnp.float32),
            pltpu.VMEM((_NBUF, _ROWS, _NUM_SNPS), jnp.float32),
            pltpu.VMEM((_NBUF, _ROWS, _NUM_SNPS), jnp.float32),
            pltpu.SemaphoreType.DMA,
            pltpu.SemaphoreType.DMA((_NBUF, 2)),
            pltpu.SemaphoreType.DMA((_NBUF, 2)),
        ],
    )(tab, idx, x)
    return out


# speedup vs baseline: 2.8567x; 2.8567x over previous
import jax
import jax.numpy as jnp
from jax.experimental import pallas as pl
from jax.experimental.pallas import tpu as pltpu

_NUM_SNPS = 100000
_NUM_IMPACTS = 16
_EMB = 16
_BATCH = 1024

_ROWS = 2000
_GRID = _NUM_SNPS // _ROWS


def _head_body(emb_ref, wpt_ref, bp_ref, gamma_ref, beta_ref, wsb_ref,
               bsbb_ref, idx_ref, sb_ref):
    h = jnp.dot(emb_ref[...], wpt_ref[...],
                preferred_element_type=jnp.float32) + bp_ref[...]
    mu = jnp.mean(h, axis=-1, keepdims=True)
    var = jnp.mean((h - mu) ** 2, axis=-1, keepdims=True)
    h = (h - mu) / jnp.sqrt(var + 1e-5) * gamma_ref[...] + beta_ref[...]
    h = jnp.maximum(h, 0.0)
    tab = jnp.dot(h, wsb_ref[...],
                  preferred_element_type=jnp.float32) + bsbb_ref[...]
    idx = idx_ref[...]
    ss = jnp.full(idx.shape, tab[0, 0] * 0.5, jnp.float32)
    bb = jnp.full(idx.shape, tab[0, 1] * 0.5, jnp.float32)
    for k in range(1, _NUM_IMPACTS):
        m = idx == k
        ss = jnp.where(m, tab[k, 0] * 0.5, ss)
        bb = jnp.where(m, tab[k, 1] * 0.5, bb)
    sb_ref[0:1, :] = ss
    sb_ref[1:2, :] = bb


def _dense_body(s_ref, b_ref, x_ref, o_ref):
    xx = x_ref[...]
    ss = s_ref[...]
    bb = b_ref[...]
    o_ref[...] = xx + xx * jnp.tanh(xx * ss + bb)


def kernel(x, impact_indices, emb, Wp, bp, gamma, beta, ws, bs, wb, bb):
    wpt = Wp.T
    wsb = jnp.concatenate([ws, wb], axis=1)
    bsbb = jnp.concatenate([bs, bb]).reshape(1, 2)
    idx = impact_indices.reshape(1, _NUM_SNPS)

    sb = pl.pallas_call(
        _head_body,
        out_shape=jax.ShapeDtypeStruct((2, _NUM_SNPS), jnp.float32),
    )(emb, wpt, bp.reshape(1, _EMB), gamma.reshape(1, _EMB),
      beta.reshape(1, _EMB), wsb, bsbb, idx)

    s_col = sb[0].reshape(_NUM_SNPS, 1)
    b_col = sb[1].reshape(_NUM_SNPS, 1)
    xt = x.T

    out_t = pl.pallas_call(
        _dense_body,
        grid=(_GRID,),
        in_specs=[
            pl.BlockSpec((_ROWS, 1), lambda j: (j, 0)),
            pl.BlockSpec((_ROWS, 1), lambda j: (j, 0)),
            pl.BlockSpec((_ROWS, _BATCH), lambda j: (j, 0)),
        ],
        out_specs=pl.BlockSpec((_ROWS, _BATCH), lambda j: (j, 0)),
        out_shape=jax.ShapeDtypeStruct((_NUM_SNPS, _BATCH), jnp.float32),
        compiler_params=pltpu.CompilerParams(
            dimension_semantics=("parallel",)),
    )(s_col, b_col, xt)
    return out_t.T
